# Initial kernel scaffold; baseline (speedup 1.0000x reference)
import jax
import jax.numpy as jnp
from jax.experimental import pallas as pl


def kernel(feature, conv_w, bn_g, bn_b, W1, b1, W2, b2, W3, b3, n1_g, n1_b, n2_g, n2_b):
    raise NotImplementedError("write your pallas kernel here")


# speedup vs baseline: 22.7272x; 22.7272x over previous
import functools

import jax
import jax.numpy as jnp
from jax.experimental import pallas as pl

H, W = 224, 224
N = H * W
HID = 64
C = 192
EPS = 1e-5
PREC = jax.lax.Precision.HIGHEST

BN1 = 16 * W
G1 = H // 16
BR = 16
GR = H // BR


def _dinv_masked(row0, rows):
    i = row0 + jax.lax.broadcasted_iota(jnp.int32, (rows, W, 1), 0)
    j = jax.lax.broadcasted_iota(jnp.int32, (rows, W, 1), 1)
    deg = ((j > 0).astype(jnp.float32) + (j < W - 1).astype(jnp.float32)
           + (i > 0).astype(jnp.float32) + (i < H - 1).astype(jnp.float32)
           + ((i > 0) & (j > 0)).astype(jnp.float32)
           + ((i < H - 1) & (j < W - 1)).astype(jnp.float32))
    dinv = jax.lax.rsqrt(deg)
    valid = (i >= 0) & (i < H)
    return jnp.where(valid, dinv, 0.0)


def _shl(x):
    z = jnp.zeros((x.shape[0], 1, x.shape[2]), x.dtype)
    return jnp.concatenate([z, x[:, : W - 1, :]], axis=1)


def _shr(x):
    z = jnp.zeros((x.shape[0], 1, x.shape[2]), x.dtype)
    return jnp.concatenate([x[:, 1:, :], z], axis=1)


def _stencil_sum(u):
    r = u.shape[0]
    mid = u[1 : r - 1]
    up = u[0 : r - 2]
    dn = u[2:r]
    return _shl(mid) + _shr(mid) + up + dn + _shl(up) + _shr(dn)


def _bn(x, stats_row, g, b):
    m = (stats_row[0:1, :] / N).reshape(1, 1, -1)
    v = (stats_row[1:2, :] / N).reshape(1, 1, -1) - m * m
    return (x - m) / jnp.sqrt(v + EPS) * g.reshape(1, 1, -1) + b.reshape(1, 1, -1)


def _acc_stats(sref, step, x2d):
    @pl.when(step == 0)
    def _():
        sref[...] = jnp.zeros_like(sref)

    sref[0:1, :] += jnp.sum(x2d, axis=0, keepdims=True)
    sref[1:2, :] += jnp.sum(x2d * x2d, axis=0, keepdims=True)


def _p1_kernel(f_ref, cw_ref, x_ref, s_ref):
    g = pl.program_id(0)
    x = jax.lax.dot_general(
        f_ref[...], cw_ref[...], (((0,), (1,)), ((), ())),
        precision=PREC, preferred_element_type=jnp.float32)
    x_ref[...] = x
    _acc_stats(s_ref, g, x)


def _p2_kernel(x_ref, s1_ref, bng_ref, bnb_ref, w1_ref, b1_ref, h_ref, s_ref):
    g = pl.program_id(0)
    x = x_ref[...][None]
    xn = jax.nn.relu(_bn(x, s1_ref[...], bng_ref[...], bnb_ref[...]))[0]
    h = jnp.dot(xn, w1_ref[...], precision=PREC,
                preferred_element_type=jnp.float32) + b1_ref[...]
    h_ref[...] = h
    _acc_stats(s_ref, g, h)


def _p3_kernel(hp_ref, hc_ref, hn_ref, s2_ref, ng_ref, nb_ref, w2_ref, b2_ref,
               o_ref, s_ref):
    g = pl.program_id(0)
    halo = 2
    row0 = g * BR - halo
    ext = jnp.concatenate(
        [hp_ref[BR - halo :], hc_ref[...], hn_ref[:halo]], axis=0)
    dm = _dinv_masked(row0, BR + 2 * halo)
    h1 = jax.nn.relu(_bn(ext, s2_ref[...], ng_ref[...], nb_ref[...]))
    acc = jnp.dot(h1[2 : BR + 2].reshape(BR * W, HID), w2_ref[0],
                  precision=PREC, preferred_element_type=jnp.float32) + b2_ref[...]
    t1 = -dm[1 : BR + 3] * _stencil_sum(dm * h1)
    acc = acc + jnp.dot(t1[1 : BR + 1].reshape(BR * W, HID), w2_ref[1],
                        precision=PREC, preferred_element_type=jnp.float32)
    t2 = (-2.0 * dm[2 : BR + 2]) * _stencil_sum(dm[1 : BR + 3] * t1) \
        - h1[2 : BR + 2]
    out = acc + jnp.dot(t2.reshape(BR * W, HID), w2_ref[2],
                        precision=PREC, preferred_element_type=jnp.float32)
    o_ref[...] = out.reshape(BR, W, HID)
    _acc_stats(s_ref, g, out)


def _p4_kernel(hp_ref, hc_ref, hn_ref, s3_ref, ng_ref, nb_ref, w3_ref, b3_ref,
               o_ref):
    g = pl.program_id(0)
    halo = 4
    row0 = g * BR - halo
    R = BR + 2 * halo
    ext = jnp.concatenate(
        [hp_ref[BR - halo :], hc_ref[...], hn_ref[:halo]], axis=0)
    dm = _dinv_masked(row0, R)

    def dot_acc(t, k):
        return jnp.dot(t.reshape(BR * W, HID), w3_ref[k], precision=PREC,
                       preferred_element_type=jnp.float32).reshape(BR, W, C)

    h2 = jax.nn.relu(_bn(ext, s3_ref[...], ng_ref[...], nb_ref[...]))
    o_ref[...] = dot_acc(h2[4 : R - 4], 0) + b3_ref[...].reshape(1, 1, C)
    t1 = -dm[1 : R - 1] * _stencil_sum(dm * h2)
    o_ref[...] += dot_acc(t1[3 : R - 5], 1)
    t2 = (-2.0 * dm[2 : R - 2]) * _stencil_sum(dm[1 : R - 1] * t1) - h2[2 : R - 2]
    o_ref[...] += dot_acc(t2[2 : R - 6], 2)
    t3 = (-2.0 * dm[3 : R - 3]) * _stencil_sum(dm[2 : R - 2] * t2) - t1[2 : R - 4]
    o_ref[...] += dot_acc(t3[1 : R - 7], 3)
    t4 = (-2.0 * dm[4 : R - 4]) * _stencil_sum(dm[3 : R - 3] * t3) - t2[2 : R - 6]
    o_ref[...] = jax.nn.sigmoid(o_ref[...] + dot_acc(t4, 4))


def _row_specs():
    return [
        pl.BlockSpec((BR, W, HID), lambda g: (jnp.maximum(g - 1, 0), 0, 0)),
        pl.BlockSpec((BR, W, HID), lambda g: (g, 0, 0)),
        pl.BlockSpec((BR, W, HID), lambda g: (jnp.minimum(g + 1, GR - 1), 0, 0)),
    ]


def _const2d(shape):
    return pl.BlockSpec(shape, lambda g: (0, 0))


@jax.jit
def kernel(feature, conv_w, bn_g, bn_b, W1, b1, W2, b2, W3, b3,
           n1_g, n1_b, n2_g, n2_b):
    fr = feature.reshape(C, N)
    bng = bn_g.reshape(1, HID)
    bnb = bn_b.reshape(1, HID)
    b1r = b1.reshape(1, HID)
    b2r = b2.reshape(1, HID)
    b3r = b3.reshape(1, C)
    n1g = n1_g.reshape(1, HID)
    n1b = n1_b.reshape(1, HID)
    n2g = n2_g.reshape(1, HID)
    n2b = n2_b.reshape(1, HID)

    x_pre, s1 = pl.pallas_call(
        _p1_kernel,
        grid=(G1,),
        in_specs=[
            pl.BlockSpec((C, BN1), lambda g: (0, g)),
            _const2d((HID, C)),
        ],
        out_specs=(
            pl.BlockSpec((BN1, HID), lambda g: (g, 0)),
            _const2d((8, HID)),
        ),
        out_shape=(
            jax.ShapeDtypeStruct((N, HID), jnp.float32),
            jax.ShapeDtypeStruct((8, HID), jnp.float32),
        ),
    )(fr, conv_w)

    h1pre, s2 = pl.pallas_call(
        _p2_kernel,
        grid=(G1,),
        in_specs=[
            pl.BlockSpec((BN1, HID), lambda g: (g, 0)),
            _const2d((8, HID)),
            _const2d((1, HID)),
            _const2d((1, HID)),
            _const2d((HID, HID)),
            _const2d((1, HID)),
        ],
        out_specs=(
            pl.BlockSpec((BN1, HID), lambda g: (g, 0)),
            _const2d((8, HID)),
        ),
        out_shape=(
            jax.ShapeDtypeStruct((N, HID), jnp.float32),
            jax.ShapeDtypeStruct((8, HID), jnp.float32),
        ),
    )(x_pre, s1, bng, bnb, W1[0], b1r)

    h1pre3 = h1pre.reshape(H, W, HID)
    h2pre, s3 = pl.pallas_call(
        _p3_kernel,
        grid=(GR,),
        in_specs=_row_specs() + [
            _const2d((8, HID)),
            _const2d((1, HID)),
            _const2d((1, HID)),
            pl.BlockSpec((3, HID, HID), lambda g: (0, 0, 0)),
            _const2d((1, HID)),
        ],
        out_specs=(
            pl.BlockSpec((BR, W, HID), lambda g: (g, 0, 0)),
            _const2d((8, HID)),
        ),
        out_shape=(
            jax.ShapeDtypeStruct((H, W, HID), jnp.float32),
            jax.ShapeDtypeStruct((8, HID), jnp.float32),
        ),
    )(h1pre3, h1pre3, h1pre3, s2, n1g, n1b, W2, b2r)

    out = pl.pallas_call(
        _p4_kernel,
        grid=(GR,),
        in_specs=_row_specs() + [
            _const2d((8, HID)),
            _const2d((1, HID)),
            _const2d((1, HID)),
            pl.BlockSpec((5, HID, C), lambda g: (0, 0, 0)),
            _const2d((1, C)),
        ],
        out_specs=pl.BlockSpec((BR, W, C), lambda g: (g, 0, 0)),
        out_shape=jax.ShapeDtypeStruct((H, W, C), jnp.float32),
    )(h2pre, h2pre, h2pre, s3, n2g, n2b, W3, b3r)

    return out.reshape(1, C, H, W)


# speedup vs baseline: 33.2851x; 1.4646x over previous
import functools

import jax
import jax.numpy as jnp
from jax.experimental import pallas as pl

H, W = 224, 224
N = H * W
HID = 64
C = 192
EPS = 1e-5
PREC = jax.lax.Precision.HIGHEST


def _split3(x):
    hi = x.astype(jnp.bfloat16)
    lo = (x - hi.astype(jnp.float32)).astype(jnp.bfloat16)
    return hi, lo


def _dot3(x, w):
    xh, xl = _split3(x)
    wh, wl = _split3(w)
    d = functools.partial(jnp.dot, preferred_element_type=jnp.float32)
    return d(xh, wl) + d(xl, wh) + d(xh, wh)

BN1 = 16 * W
G1 = H // 16
BR = 16
GR = H // BR


def _dinv_masked(row0, rows):
    i = row0 + jax.lax.broadcasted_iota(jnp.int32, (rows, W, 1), 0)
    j = jax.lax.broadcasted_iota(jnp.int32, (rows, W, 1), 1)
    deg = ((j > 0).astype(jnp.float32) + (j < W - 1).astype(jnp.float32)
           + (i > 0).astype(jnp.float32) + (i < H - 1).astype(jnp.float32)
           + ((i > 0) & (j > 0)).astype(jnp.float32)
           + ((i < H - 1) & (j < W - 1)).astype(jnp.float32))
    dinv = jax.lax.rsqrt(deg)
    valid = (i >= 0) & (i < H)
    return jnp.where(valid, dinv, 0.0)


def _shl(x):
    z = jnp.zeros((x.shape[0], 1, x.shape[2]), x.dtype)
    return jnp.concatenate([z, x[:, : W - 1, :]], axis=1)


def _shr(x):
    z = jnp.zeros((x.shape[0], 1, x.shape[2]), x.dtype)
    return jnp.concatenate([x[:, 1:, :], z], axis=1)


def _stencil_sum(u):
    r = u.shape[0]
    mid = u[1 : r - 1]
    up = u[0 : r - 2]
    dn = u[2:r]
    return _shl(mid) + _shr(mid) + up + dn + _shl(up) + _shr(dn)


def _bn(x, stats_row, g, b):
    m = (stats_row[0:1, :] / N).reshape(1, 1, -1)
    v = (stats_row[1:2, :] / N).reshape(1, 1, -1) - m * m
    return (x - m) / jnp.sqrt(v + EPS) * g.reshape(1, 1, -1) + b.reshape(1, 1, -1)


def _acc_stats(sref, step, x2d):
    @pl.when(step == 0)
    def _():
        sref[...] = jnp.zeros_like(sref)

    sref[0:1, :] += jnp.sum(x2d, axis=0, keepdims=True)
    sref[1:2, :] += jnp.sum(x2d * x2d, axis=0, keepdims=True)


def _p1_kernel(f_ref, cw_ref, x_ref, s_ref):
    g = pl.program_id(0)
    fh, fl = _split3(f_ref[...])
    ch, cl = _split3(cw_ref[...])
    dg = functools.partial(
        jax.lax.dot_general,
        dimension_numbers=(((0,), (1,)), ((), ())),
        preferred_element_type=jnp.float32)
    x = dg(fh, cl) + dg(fl, ch) + dg(fh, ch)
    x_ref[...] = x
    _acc_stats(s_ref, g, x)


def _p2_kernel(x_ref, s1_ref, bng_ref, bnb_ref, w1_ref, b1_ref, h_ref, s_ref):
    g = pl.program_id(0)
    x = x_ref[...][None]
    xn = jax.nn.relu(_bn(x, s1_ref[...], bng_ref[...], bnb_ref[...]))[0]
    h = _dot3(xn, w1_ref[...]) + b1_ref[...]
    h_ref[...] = h
    _acc_stats(s_ref, g, h)


def _p3_kernel(hp_ref, hc_ref, hn_ref, s2_ref, ng_ref, nb_ref, w2_ref, b2_ref,
               o_ref, s_ref):
    g = pl.program_id(0)
    halo = 2
    row0 = g * BR - halo
    ext = jnp.concatenate(
        [hp_ref[BR - halo :], hc_ref[...], hn_ref[:halo]], axis=0)
    dm = _dinv_masked(row0, BR + 2 * halo)
    h1 = jax.nn.relu(_bn(ext, s2_ref[...], ng_ref[...], nb_ref[...]))
    acc = _dot3(h1[2 : BR + 2].reshape(BR * W, HID), w2_ref[0]) + b2_ref[...]
    t1 = -dm[1 : BR + 3] * _stencil_sum(dm * h1)
    acc = acc + _dot3(t1[1 : BR + 1].reshape(BR * W, HID), w2_ref[1])
    t2 = (-2.0 * dm[2 : BR + 2]) * _stencil_sum(dm[1 : BR + 3] * t1) \
        - h1[2 : BR + 2]
    out = acc + _dot3(t2.reshape(BR * W, HID), w2_ref[2])
    o_ref[...] = out.reshape(BR, W, HID)
    _acc_stats(s_ref, g, out)


def _p4_kernel(hp_ref, hc_ref, hn_ref, s3_ref, ng_ref, nb_ref, w3_ref, b3_ref,
               o_ref):
    g = pl.program_id(0)
    halo = 4
    row0 = g * BR - halo
    R = BR + 2 * halo
    ext = jnp.concatenate(
        [hp_ref[BR - halo :], hc_ref[...], hn_ref[:halo]], axis=0)
    dm = _dinv_masked(row0, R)

    def dot_acc(t, k):
        return _dot3(t.reshape(BR * W, HID), w3_ref[k]).reshape(BR, W, C)

    h2 = jax.nn.relu(_bn(ext, s3_ref[...], ng_ref[...], nb_ref[...]))
    o_ref[...] = dot_acc(h2[4 : R - 4], 0) + b3_ref[...].reshape(1, 1, C)
    t1 = -dm[1 : R - 1] * _stencil_sum(dm * h2)
    o_ref[...] += dot_acc(t1[3 : R - 5], 1)
    t2 = (-2.0 * dm[2 : R - 2]) * _stencil_sum(dm[1 : R - 1] * t1) - h2[2 : R - 2]
    o_ref[...] += dot_acc(t2[2 : R - 6], 2)
    t3 = (-2.0 * dm[3 : R - 3]) * _stencil_sum(dm[2 : R - 2] * t2) - t1[2 : R - 4]
    o_ref[...] += dot_acc(t3[1 : R - 7], 3)
    t4 = (-2.0 * dm[4 : R - 4]) * _stencil_sum(dm[3 : R - 3] * t3) - t2[2 : R - 6]
    o_ref[...] = jax.nn.sigmoid(o_ref[...] + dot_acc(t4, 4))


def _row_specs():
    return [
        pl.BlockSpec((BR, W, HID), lambda g: (jnp.maximum(g - 1, 0), 0, 0)),
        pl.BlockSpec((BR, W, HID), lambda g: (g, 0, 0)),
        pl.BlockSpec((BR, W, HID), lambda g: (jnp.minimum(g + 1, GR - 1), 0, 0)),
    ]


def _const2d(shape):
    return pl.BlockSpec(shape, lambda g: (0, 0))


@jax.jit
def kernel(feature, conv_w, bn_g, bn_b, W1, b1, W2, b2, W3, b3,
           n1_g, n1_b, n2_g, n2_b):
    fr = feature.reshape(C, N)
    bng = bn_g.reshape(1, HID)
    bnb = bn_b.reshape(1, HID)
    b1r = b1.reshape(1, HID)
    b2r = b2.reshape(1, HID)
    b3r = b3.reshape(1, C)
    n1g = n1_g.reshape(1, HID)
    n1b = n1_b.reshape(1, HID)
    n2g = n2_g.reshape(1, HID)
    n2b = n2_b.reshape(1, HID)

    x_pre, s1 = pl.pallas_call(
        _p1_kernel,
        grid=(G1,),
        in_specs=[
            pl.BlockSpec((C, BN1), lambda g: (0, g)),
            _const2d((HID, C)),
        ],
        out_specs=(
            pl.BlockSpec((BN1, HID), lambda g: (g, 0)),
            _const2d((8, HID)),
        ),
        out_shape=(
            jax.ShapeDtypeStruct((N, HID), jnp.float32),
            jax.ShapeDtypeStruct((8, HID), jnp.float32),
        ),
    )(fr, conv_w)

    h1pre, s2 = pl.pallas_call(
        _p2_kernel,
        grid=(G1,),
        in_specs=[
            pl.BlockSpec((BN1, HID), lambda g: (g, 0)),
            _const2d((8, HID)),
            _const2d((1, HID)),
            _const2d((1, HID)),
            _const2d((HID, HID)),
            _const2d((1, HID)),
        ],
        out_specs=(
            pl.BlockSpec((BN1, HID), lambda g: (g, 0)),
            _const2d((8, HID)),
        ),
        out_shape=(
            jax.ShapeDtypeStruct((N, HID), jnp.float32),
            jax.ShapeDtypeStruct((8, HID), jnp.float32),
        ),
    )(x_pre, s1, bng, bnb, W1[0], b1r)

    h1pre3 = h1pre.reshape(H, W, HID)
    h2pre, s3 = pl.pallas_call(
        _p3_kernel,
        grid=(GR,),
        in_specs=_row_specs() + [
            _const2d((8, HID)),
            _const2d((1, HID)),
            _const2d((1, HID)),
            pl.BlockSpec((3, HID, HID), lambda g: (0, 0, 0)),
            _const2d((1, HID)),
        ],
        out_specs=(
            pl.BlockSpec((BR, W, HID), lambda g: (g, 0, 0)),
            _const2d((8, HID)),
        ),
        out_shape=(
            jax.ShapeDtypeStruct((H, W, HID), jnp.float32),
            jax.ShapeDtypeStruct((8, HID), jnp.float32),
        ),
    )(h1pre3, h1pre3, h1pre3, s2, n1g, n1b, W2, b2r)

    out = pl.pallas_call(
        _p4_kernel,
        grid=(GR,),
        in_specs=_row_specs() + [
            _const2d((8, HID)),
            _const2d((1, HID)),
            _const2d((1, HID)),
            pl.BlockSpec((5, HID, C), lambda g: (0, 0, 0)),
            _const2d((1, C)),
        ],
        out_specs=pl.BlockSpec((BR, W, C), lambda g: (g, 0, 0)),
        out_shape=jax.ShapeDtypeStruct((H, W, C), jnp.float32),
    )(h2pre, h2pre, h2pre, s3, n2g, n2b, W3, b3r)

    return out.reshape(1, C, H, W)


# speedup vs baseline: 37.1872x; 1.1172x over previous
import functools

import jax
import jax.numpy as jnp
from jax.experimental import pallas as pl

H, W = 224, 224
N = H * W
HID = 64
C = 192
EPS = 1e-5
PREC = jax.lax.Precision.HIGHEST


def _split3(x):
    hi = x.astype(jnp.bfloat16)
    lo = (x - hi.astype(jnp.float32)).astype(jnp.bfloat16)
    return hi, lo


def _dot3(x, w):
    xh, xl = _split3(x)
    wh, wl = _split3(w)
    d = functools.partial(jnp.dot, preferred_element_type=jnp.float32)
    return d(xh, wl) + d(xl, wh) + d(xh, wh)

BN1 = 16 * W
G1 = H // 16
BR = 16
GR = H // BR


def _dinv_masked(row0, rows):
    i = row0 + jax.lax.broadcasted_iota(jnp.int32, (rows, W, 1), 0)
    j = jax.lax.broadcasted_iota(jnp.int32, (rows, W, 1), 1)
    deg = ((j > 0).astype(jnp.float32) + (j < W - 1).astype(jnp.float32)
           + (i > 0).astype(jnp.float32) + (i < H - 1).astype(jnp.float32)
           + ((i > 0) & (j > 0)).astype(jnp.float32)
           + ((i < H - 1) & (j < W - 1)).astype(jnp.float32))
    dinv = jax.lax.rsqrt(deg)
    valid = (i >= 0) & (i < H)
    return jnp.where(valid, dinv, 0.0)


def _shl(x):
    z = jnp.zeros((x.shape[0], 1, x.shape[2]), x.dtype)
    return jnp.concatenate([z, x[:, : W - 1, :]], axis=1)


def _shr(x):
    z = jnp.zeros((x.shape[0], 1, x.shape[2]), x.dtype)
    return jnp.concatenate([x[:, 1:, :], z], axis=1)


def _stencil_sum(u):
    r = u.shape[0]
    mid = u[1 : r - 1]
    up = u[0 : r - 2]
    dn = u[2:r]
    return _shl(mid) + _shr(mid) + up + dn + _shl(up) + _shr(dn)


def _bn(x, stats_row, g, b):
    m = (stats_row[0:1, :] / N).reshape(1, 1, -1)
    v = (stats_row[1:2, :] / N).reshape(1, 1, -1) - m * m
    return (x - m) / jnp.sqrt(v + EPS) * g.reshape(1, 1, -1) + b.reshape(1, 1, -1)


def _acc_stats(sref, step, x2d):
    @pl.when(step == 0)
    def _():
        sref[...] = jnp.zeros_like(sref)

    sref[0:1, :] += jnp.sum(x2d, axis=0, keepdims=True)
    sref[1:2, :] += jnp.sum(x2d * x2d, axis=0, keepdims=True)


def _p1_kernel(f_ref, cw_ref, x_ref, s_ref):
    g = pl.program_id(0)
    fh, fl = _split3(f_ref[...])
    ch, cl = _split3(cw_ref[...])
    dg = functools.partial(
        jax.lax.dot_general,
        dimension_numbers=(((0,), (1,)), ((), ())),
        preferred_element_type=jnp.float32)
    x = dg(fh, cl) + dg(fl, ch) + dg(fh, ch)
    x_ref[...] = x
    _acc_stats(s_ref, g, x)


def _p2_kernel(x_ref, s1_ref, bng_ref, bnb_ref, w1_ref, b1_ref, h_ref, s_ref):
    g = pl.program_id(0)
    x = x_ref[...][None]
    xn = jax.nn.relu(_bn(x, s1_ref[...], bng_ref[...], bnb_ref[...]))[0]
    h = _dot3(xn, w1_ref[...]) + b1_ref[...]
    h_ref[...] = h
    _acc_stats(s_ref, g, h)


def _p3_kernel(hp_ref, hc_ref, hn_ref, s2_ref, ng_ref, nb_ref, w2_ref, b2_ref,
               o_ref, s_ref):
    g = pl.program_id(0)
    halo = 2
    row0 = g * BR - halo
    ext = jnp.concatenate(
        [hp_ref[BR - halo :], hc_ref[...], hn_ref[:halo]], axis=0)
    dm = _dinv_masked(row0, BR + 2 * halo)
    h1 = jax.nn.relu(_bn(ext, s2_ref[...], ng_ref[...], nb_ref[...]))
    t1 = -dm[1 : BR + 3] * _stencil_sum(dm * h1)
    t2 = (-2.0 * dm[2 : BR + 2]) * _stencil_sum(dm[1 : BR + 3] * t1) \
        - h1[2 : BR + 2]
    lhs = jnp.concatenate(
        [h1[2 : BR + 2], t1[1 : BR + 1], t2], axis=-1).reshape(BR * W, 3 * HID)
    out = _dot3(lhs, w2_ref[...]) + b2_ref[...]
    o_ref[...] = out.reshape(BR, W, HID)
    _acc_stats(s_ref, g, out)


def _p4_kernel(hp_ref, hc_ref, hn_ref, s3_ref, ng_ref, nb_ref, w3_ref, b3_ref,
               o_ref):
    g = pl.program_id(0)
    halo = 4
    row0 = g * BR - halo
    R = BR + 2 * halo
    ext = jnp.concatenate(
        [hp_ref[BR - halo :], hc_ref[...], hn_ref[:halo]], axis=0)
    dm = _dinv_masked(row0, R)

    h2 = jax.nn.relu(_bn(ext, s3_ref[...], ng_ref[...], nb_ref[...]))
    t1 = -dm[1 : R - 1] * _stencil_sum(dm * h2)
    t2 = (-2.0 * dm[2 : R - 2]) * _stencil_sum(dm[1 : R - 1] * t1) - h2[2 : R - 2]
    t3 = (-2.0 * dm[3 : R - 3]) * _stencil_sum(dm[2 : R - 2] * t2) - t1[2 : R - 4]
    t4 = (-2.0 * dm[4 : R - 4]) * _stencil_sum(dm[3 : R - 3] * t3) - t2[2 : R - 6]
    lhs = jnp.concatenate(
        [h2[4 : R - 4], t1[3 : R - 5], t2[2 : R - 6], t3[1 : R - 7], t4],
        axis=-1).reshape(BR * W, 5 * HID)
    out = _dot3(lhs, w3_ref[...]) + b3_ref[...]
    o_ref[...] = jax.nn.sigmoid(out).reshape(BR, W, C)


def _row_specs():
    return [
        pl.BlockSpec((BR, W, HID), lambda g: (jnp.maximum(g - 1, 0), 0, 0)),
        pl.BlockSpec((BR, W, HID), lambda g: (g, 0, 0)),
        pl.BlockSpec((BR, W, HID), lambda g: (jnp.minimum(g + 1, GR - 1), 0, 0)),
    ]


def _const2d(shape):
    return pl.BlockSpec(shape, lambda g: (0, 0))


@jax.jit
def kernel(feature, conv_w, bn_g, bn_b, W1, b1, W2, b2, W3, b3,
           n1_g, n1_b, n2_g, n2_b):
    fr = feature.reshape(C, N)
    bng = bn_g.reshape(1, HID)
    bnb = bn_b.reshape(1, HID)
    b1r = b1.reshape(1, HID)
    b2r = b2.reshape(1, HID)
    b3r = b3.reshape(1, C)
    n1g = n1_g.reshape(1, HID)
    n1b = n1_b.reshape(1, HID)
    n2g = n2_g.reshape(1, HID)
    n2b = n2_b.reshape(1, HID)

    x_pre, s1 = pl.pallas_call(
        _p1_kernel,
        grid=(G1,),
        in_specs=[
            pl.BlockSpec((C, BN1), lambda g: (0, g)),
            _const2d((HID, C)),
        ],
        out_specs=(
            pl.BlockSpec((BN1, HID), lambda g: (g, 0)),
            _const2d((8, HID)),
        ),
        out_shape=(
            jax.ShapeDtypeStruct((N, HID), jnp.float32),
            jax.ShapeDtypeStruct((8, HID), jnp.float32),
        ),
    )(fr, conv_w)

    h1pre, s2 = pl.pallas_call(
        _p2_kernel,
        grid=(G1,),
        in_specs=[
            pl.BlockSpec((BN1, HID), lambda g: (g, 0)),
            _const2d((8, HID)),
            _const2d((1, HID)),
            _const2d((1, HID)),
            _const2d((HID, HID)),
            _const2d((1, HID)),
        ],
        out_specs=(
            pl.BlockSpec((BN1, HID), lambda g: (g, 0)),
            _const2d((8, HID)),
        ),
        out_shape=(
            jax.ShapeDtypeStruct((N, HID), jnp.float32),
            jax.ShapeDtypeStruct((8, HID), jnp.float32),
        ),
    )(x_pre, s1, bng, bnb, W1[0], b1r)

    h1pre3 = h1pre.reshape(H, W, HID)
    h2pre, s3 = pl.pallas_call(
        _p3_kernel,
        grid=(GR,),
        in_specs=_row_specs() + [
            _const2d((8, HID)),
            _const2d((1, HID)),
            _const2d((1, HID)),
            _const2d((3 * HID, HID)),
            _const2d((1, HID)),
        ],
        out_specs=(
            pl.BlockSpec((BR, W, HID), lambda g: (g, 0, 0)),
            _const2d((8, HID)),
        ),
        out_shape=(
            jax.ShapeDtypeStruct((H, W, HID), jnp.float32),
            jax.ShapeDtypeStruct((8, HID), jnp.float32),
        ),
    )(h1pre3, h1pre3, h1pre3, s2, n1g, n1b, W2.reshape(3 * HID, HID), b2r)

    out = pl.pallas_call(
        _p4_kernel,
        grid=(GR,),
        in_specs=_row_specs() + [
            _const2d((8, HID)),
            _const2d((1, HID)),
            _const2d((1, HID)),
            _const2d((5 * HID, C)),
            _const2d((1, C)),
        ],
        out_specs=pl.BlockSpec((BR, W, C), lambda g: (g, 0, 0)),
        out_shape=jax.ShapeDtypeStruct((H, W, C), jnp.float32),
    )(h2pre, h2pre, h2pre, s3, n2g, n2b, W3.reshape(5 * HID, C), b3r)

    return out.reshape(1, C, H, W)
